# Initial kernel scaffold; baseline (speedup 1.0000x reference)
import jax
import jax.numpy as jnp
from jax.experimental import pallas as pl


def kernel(x, edge_index, W1, b1, W2, b2):
    raise NotImplementedError("write your pallas kernel here")


# speedup vs baseline: 9.7034x; 9.7034x over previous
import functools

import jax
import jax.numpy as jnp
from jax import lax
from jax.experimental import pallas as pl
from jax.experimental.pallas import tpu as pltpu
from jax.experimental.pallas import tpu_sc as plsc

N = 10000
D = 128
E = 320000

NC = 2
NS = 16
NW = NC * NS

NPAD = 10240
ROWS_PER_TILE = NPAD // NS

C = 128
KJ = 80
EROWS = NW * KJ
EPAD = EROWS * C

DEGW = 16

_MESH = plsc.VectorSubcoreMesh(core_axis_name="c", subcore_axis_name="s")


@functools.partial(
    pl.kernel,
    mesh=_MESH,
    out_type=jax.ShapeDtypeStruct((NC, NPAD, DEGW), jnp.float32),
    scratch_types=[
        pltpu.VMEM((KJ, C), jnp.int32),
        pltpu.VMEM((C, DEGW), jnp.float32),
        pltpu.VMEM_SHARED((NPAD, DEGW), jnp.float32),
    ],
)
def _deg_kernel(dst_hbm, ones_hbm, z16_hbm, out_hbm, dst_v, ones_v, acc):
    c = lax.axis_index("c")
    s = lax.axis_index("s")
    wid = s * NC + c
    rs = s * ROWS_PER_TILE
    pltpu.sync_copy(z16_hbm, acc.at[pl.ds(rs, ROWS_PER_TILE)])
    pltpu.sync_copy(ones_hbm, ones_v)
    pltpu.sync_copy(dst_hbm.at[pl.ds(wid * KJ, KJ)], dst_v)
    plsc.subcore_barrier()

    def body(j, carry):
        pltpu.sync_copy(ones_v, acc.at[dst_v.at[j]], add=True)
        return carry

    lax.fori_loop(0, KJ, body, 0)
    plsc.subcore_barrier()
    pltpu.sync_copy(acc.at[pl.ds(rs, ROWS_PER_TILE)],
                    out_hbm.at[c, pl.ds(rs, ROWS_PER_TILE)])


@functools.partial(
    pl.kernel,
    mesh=_MESH,
    out_type=jax.ShapeDtypeStruct((NC, NPAD, D), jnp.float32),
    scratch_types=[
        pltpu.VMEM((KJ, C), jnp.int32),
        pltpu.VMEM((KJ, C), jnp.int32),
        pltpu.VMEM((C, D), jnp.float32),
        pltpu.VMEM_SHARED((NPAD, D), jnp.float32),
        pltpu.SemaphoreType.DMA,
    ],
)
def _agg_kernel(g_hbm, src_hbm, dst_hbm, z_hbm, out_hbm,
                src_v, dst_v, buf, acc, gsem):
    c = lax.axis_index("c")
    s = lax.axis_index("s")
    wid = s * NC + c
    rs = s * ROWS_PER_TILE

    @pl.when(c == 0)
    def _():
        pltpu.sync_copy(g_hbm.at[pl.ds(rs, ROWS_PER_TILE)],
                        acc.at[pl.ds(rs, ROWS_PER_TILE)])

    @pl.when(c != 0)
    def _():
        pltpu.sync_copy(z_hbm, acc.at[pl.ds(rs, ROWS_PER_TILE)])

    pltpu.sync_copy(src_hbm.at[pl.ds(wid * KJ, KJ)], src_v)
    pltpu.sync_copy(dst_hbm.at[pl.ds(wid * KJ, KJ)], dst_v)
    plsc.subcore_barrier()

    def body(j, carry):
        pltpu.async_copy(g_hbm.at[src_v.at[j]], buf, gsem).wait()
        pltpu.sync_copy(buf, acc.at[dst_v.at[j]], add=True)
        return carry

    lax.fori_loop(0, KJ, body, 0)
    plsc.subcore_barrier()
    pltpu.sync_copy(acc.at[pl.ds(rs, ROWS_PER_TILE)],
                    out_hbm.at[c, pl.ds(rs, ROWS_PER_TILE)])


_R = 512


def _dis(degp_ref):
    deg = degp_ref[0][:, :1] + degp_ref[1][:, :1] + 1.0
    return lax.rsqrt(deg)


def _prep_body(degp_ref, x_ref, w_ref, g_ref):
    dis = _dis(degp_ref)
    g_ref[...] = dis * jnp.dot(x_ref[...], w_ref[...],
                               preferred_element_type=jnp.float32)


def _mid_body(p_ref, degp_ref, w_ref, b_ref, g2_ref):
    dis = _dis(degp_ref)
    h = jnp.maximum(dis * (p_ref[0] + p_ref[1]) + b_ref[...], 0.0)
    g2_ref[...] = dis * jnp.dot(h, w_ref[...],
                                preferred_element_type=jnp.float32)


def _final_body(p_ref, degp_ref, b_ref, out_ref):
    dis = _dis(degp_ref)
    out_ref[...] = jnp.maximum(dis * (p_ref[0] + p_ref[1]) + b_ref[...], 0.0)


_DEGP_SPEC = pl.BlockSpec((NC, _R, DEGW), lambda i: (0, i, 0))
_P_SPEC = pl.BlockSpec((NC, _R, D), lambda i: (0, i, 0))
_ROW_SPEC = pl.BlockSpec((_R, D), lambda i: (i, 0))
_W_SPEC = pl.BlockSpec((D, D), lambda i: (0, 0))
_B_SPEC = pl.BlockSpec((1, D), lambda i: (0, 0))
_GRID = NPAD // _R

_prep_call = pl.pallas_call(
    _prep_body,
    grid=(_GRID,),
    in_specs=[_DEGP_SPEC, _ROW_SPEC, _W_SPEC],
    out_specs=_ROW_SPEC,
    out_shape=jax.ShapeDtypeStruct((NPAD, D), jnp.float32),
)

_mid_call = pl.pallas_call(
    _mid_body,
    grid=(_GRID,),
    in_specs=[_P_SPEC, _DEGP_SPEC, _W_SPEC, _B_SPEC],
    out_specs=_ROW_SPEC,
    out_shape=jax.ShapeDtypeStruct((NPAD, D), jnp.float32),
)

_final_call = pl.pallas_call(
    _final_body,
    grid=(_GRID,),
    in_specs=[_P_SPEC, _DEGP_SPEC, _B_SPEC],
    out_specs=_ROW_SPEC,
    out_shape=jax.ShapeDtypeStruct((NPAD, D), jnp.float32),
)


def kernel(x, edge_index, W1, b1, W2, b2):
    src = edge_index[0]
    dst = edge_index[1]
    pad = jnp.full((EPAD - E,), N, dtype=jnp.int32)
    src2 = jnp.concatenate([src, pad]).reshape(EROWS, C)
    dst2 = jnp.concatenate([dst, pad]).reshape(EROWS, C)
    xpad = jnp.zeros((NPAD, D), jnp.float32).at[:N].set(x)

    ones16 = jnp.ones((C, DEGW), jnp.float32)
    z16 = jnp.zeros((ROWS_PER_TILE, DEGW), jnp.float32)
    z640 = jnp.zeros((ROWS_PER_TILE, D), jnp.float32)

    degp = _deg_kernel(dst2, ones16, z16)
    g1 = _prep_call(degp, xpad, W1)
    p1 = _agg_kernel(g1, src2, dst2, z640)
    g2 = _mid_call(p1, degp, W2, b1.reshape(1, D))
    p2 = _agg_kernel(g2, src2, dst2, z640)
    out = _final_call(p2, degp, b2.reshape(1, D))
    return out[:N]
